# baseline (speedup 1.0000x reference)
import functools

import jax
import jax.numpy as jnp
from jax import lax
from jax.experimental import pallas as pl
from jax.experimental.pallas import tpu as pltpu
from jax.experimental.pallas import tpu_sc as plsc

N_ROWS = 64
N_COLS = 1_000_000
K = 8
TILE = 1024
NT = (N_COLS + TILE - 1) // TILE
NT_PAD = 1024
GRAN = 16
G_PER_TILE = TILE // GRAN
N_TABLE = N_ROWS * N_COLS // GRAN
N_IDX = N_ROWS * K * G_PER_TILE
NEG = float("-inf")
IMAX = 2**31 - 1

_GUMBEL_BOX = []


def _gumbel_const():
    if not _GUMBEL_BOX:
        def draw():
            return jax.random.gumbel(
                jax.random.key(42), (N_ROWS, N_COLS), jnp.float32
            )

        try:
            with jax.ensure_compile_time_eval():
                _GUMBEL_BOX.append(draw())
        except Exception:
            return draw()
    return _GUMBEL_BOX[0]


def _tile_max_body(p_ref, g_ref, out_ref):
    t = pl.program_id(0)
    z = jnp.log(p_ref[...]) + g_ref[...]
    col = lax.broadcasted_iota(jnp.int32, (N_ROWS, TILE), 1) + t * TILE
    z = jnp.where(col < N_COLS, z, NEG)
    out_ref[...] = jnp.max(z, axis=1, keepdims=True).reshape(1, N_ROWS, 1)


def _tile_max(probs, g):
    return pl.pallas_call(
        _tile_max_body,
        grid=(NT,),
        in_specs=[
            pl.BlockSpec((N_ROWS, TILE), lambda t: (0, t)),
            pl.BlockSpec((N_ROWS, TILE), lambda t: (0, t)),
        ],
        out_specs=pl.BlockSpec((1, N_ROWS, 1), lambda t: (t, 0, 0)),
        out_shape=jax.ShapeDtypeStruct((NT_PAD, N_ROWS, 1), jnp.float32),
    )(probs, g)


def _select_body(tmax_ref, sel_ref, idx_ref):
    x = tmax_ref[...]
    col = lax.broadcasted_iota(jnp.int32, (N_ROWS, NT_PAD), 1)
    x = jnp.where(col < NT, x, NEG)
    sel_cols = []
    for _ in range(K):
        m = jnp.max(x, axis=1, keepdims=True)
        cand = jnp.where(x == m, col, IMAX)
        t_sel = jnp.min(cand, axis=1, keepdims=True)
        sel_cols.append(t_sel)
        x = jnp.where(col == t_sel, NEG, x)
    sel_ref[...] = jnp.concatenate(sel_cols, axis=1)

    col2 = lax.broadcasted_iota(jnp.int32, (N_ROWS, K * G_PER_TILE), 1)
    kk = col2 >> 6
    j = col2 & (G_PER_TILE - 1)
    sel_k = jnp.zeros((N_ROWS, K * G_PER_TILE), jnp.int32)
    for k_i in range(K):
        sel_k = jnp.where(kk == k_i, sel_cols[k_i], sel_k)
    row = lax.broadcasted_iota(jnp.int32, (N_ROWS, K * G_PER_TILE), 0)
    gidx = row * (N_COLS // GRAN) + sel_k * G_PER_TILE + j
    idx_ref[...] = jnp.minimum(gidx, N_TABLE - 1)


def _select(tmax):
    return pl.pallas_call(
        _select_body,
        out_shape=(
            jax.ShapeDtypeStruct((N_ROWS, K), jnp.int32),
            jax.ShapeDtypeStruct((N_ROWS, K * G_PER_TILE), jnp.int32),
        ),
    )(tmax)


_NW = 32
_IDX_ROWS = N_IDX // 128
_RPW = _IDX_ROWS // _NW


def _sc_gather(p_tab, g_tab, idx):
    mesh = plsc.VectorSubcoreMesh(core_axis_name="c", subcore_axis_name="s")

    @functools.partial(
        pl.kernel,
        mesh=mesh,
        compiler_params=pltpu.CompilerParams(use_tc_tiling_on_sc=False),
        out_type=(
            jax.ShapeDtypeStruct((_IDX_ROWS, 128, GRAN), jnp.float32),
            jax.ShapeDtypeStruct((_IDX_ROWS, 128, GRAN), jnp.float32),
        ),
        scratch_types=[
            pltpu.VMEM((_RPW, 128), jnp.int32),
            pltpu.VMEM((_RPW, 128, GRAN), jnp.float32),
            pltpu.SemaphoreType.DMA,
        ],
    )
    def gather_kernel(p_hbm, g_hbm, idx_hbm, p_out, g_out, idx_v, buf, sem):
        wid = lax.axis_index("s") * 2 + lax.axis_index("c")
        base = wid * _RPW
        pltpu.sync_copy(idx_hbm.at[pl.ds(base, _RPW)], idx_v)
        for src, dst in ((p_hbm, p_out), (g_hbm, g_out)):
            copies = [
                pltpu.async_copy(src.at[idx_v.at[r]], buf.at[r], sem)
                for r in range(_RPW)
            ]
            for c in copies:
                c.wait()
            pltpu.sync_copy(buf, dst.at[pl.ds(base, _RPW)])

    return gather_kernel(p_tab, g_tab, idx)


def _final_body(p_ref, g_ref, sel_ref, out_ref):
    col = lax.broadcasted_iota(jnp.int32, (N_ROWS, K * TILE), 1)
    kk = col >> 10
    off = col & (TILE - 1)
    sel = sel_ref[...]
    sel_k = jnp.zeros((N_ROWS, K * TILE), jnp.int32)
    for k_i in range(K):
        sel_k = jnp.where(kk == k_i, sel[:, k_i : k_i + 1], sel_k)
    gcol = sel_k * TILE + off
    z = jnp.log(p_ref[...]) + g_ref[...]
    z = jnp.where(gcol < N_COLS, z, NEG)
    outs = []
    for _ in range(K):
        m = jnp.max(z, axis=1, keepdims=True)
        cand = jnp.where(z == m, gcol, IMAX)
        gmin = jnp.min(cand, axis=1, keepdims=True)
        outs.append(gmin)
        z = jnp.where(gcol == gmin, NEG, z)
    out_ref[...] = jnp.concatenate(outs, axis=1)


def _final(p_gath, g_gath, sel):
    return pl.pallas_call(
        _final_body,
        out_shape=jax.ShapeDtypeStruct((N_ROWS, K), jnp.int32),
    )(p_gath, g_gath, sel)


def kernel(probs):
    g = _gumbel_const()
    tmax3 = _tile_max(probs, g)
    sel, gidx = _select(tmax3.reshape(NT_PAD, N_ROWS).T)
    p_tab = probs.reshape(N_TABLE, GRAN)
    g_tab = g.reshape(N_TABLE, GRAN)
    p_gath, g_gath = _sc_gather(p_tab, g_tab, gidx.reshape(_IDX_ROWS, 128))
    return _final(
        p_gath.reshape(N_ROWS, K * TILE), g_gath.reshape(N_ROWS, K * TILE), sel
    )
